# Initial kernel scaffold; baseline (speedup 1.0000x reference)
import jax
import jax.numpy as jnp
from jax.experimental import pallas as pl


def kernel(x, adj, W_s1, b_s1, W_s2, b_s2, W_p1, b_p1, W_p2, b_p2, enc_mask_token):
    raise NotImplementedError("write your pallas kernel here")


# speedup vs baseline: 1.1623x; 1.1623x over previous
import functools

import jax
import jax.numpy as jnp
from jax.experimental import pallas as pl
from jax.experimental.pallas import tpu as pltpu


def _layer1_kernel(adj_ref, x_ref, w1_ref, b1_ref, w2_ref, o_ref, acc_ref,
                   *, bk, n_total):
    k = pl.program_id(1)
    nk = pl.num_programs(1)

    @pl.when(k == 0)
    def _():
        acc_ref[...] = jnp.zeros_like(acc_ref)

    a = adj_ref[...]
    cols = jax.lax.broadcasted_iota(jnp.int32, a.shape, 1)
    a = jnp.where(cols < (n_total - k * bk), a, 0.0)
    acc_ref[...] += jnp.dot(a.astype(jnp.bfloat16),
                            x_ref[...].astype(jnp.bfloat16),
                            preferred_element_type=jnp.float32)

    @pl.when(k == nk - 1)
    def _():
        z = acc_ref[...]
        h = jnp.dot(z.astype(jnp.bfloat16), w1_ref[...].astype(jnp.bfloat16),
                    preferred_element_type=jnp.float32) + b1_ref[...]
        h = jnp.maximum(h, 0.0)
        o_ref[...] = jnp.dot(h.astype(jnp.bfloat16),
                             w2_ref[...].astype(jnp.bfloat16),
                             preferred_element_type=jnp.float32)


def _layer2_kernel(adj_ref, p_ref, b_ref, o_ref, acc_ref, *, bk, n_total):
    k = pl.program_id(1)
    nk = pl.num_programs(1)

    @pl.when(k == 0)
    def _():
        acc_ref[...] = jnp.zeros_like(acc_ref)

    a = adj_ref[...]
    cols = jax.lax.broadcasted_iota(jnp.int32, a.shape, 1)
    a = jnp.where(cols < (n_total - k * bk), a, 0.0)
    acc_ref[...] += jnp.dot(a.astype(jnp.bfloat16),
                            p_ref[...].astype(jnp.bfloat16),
                            preferred_element_type=jnp.float32)

    @pl.when(k == nk - 1)
    def _():
        o_ref[...] = acc_ref[...] + b_ref[...]


def kernel(x, adj, W_s1, b_s1, W_s2, b_s2, W_p1, b_p1, W_p2, b_p2,
           enc_mask_token):
    del enc_mask_token
    n, ft = x.shape
    hid = W_s1.shape[1]
    com = W_s2.shape[1]

    bm = 400
    bk = 2048
    grid_m = pl.cdiv(n, bm)
    grid_k = pl.cdiv(n, bk)
    k_pad = grid_k * bk

    w1 = jnp.concatenate([W_s1, W_p1], axis=1)
    b1 = jnp.concatenate([b_s1, b_p1]).reshape(1, 2 * hid)
    w2 = jnp.zeros((2 * hid, 2 * com), jnp.float32)
    w2 = w2.at[:hid, :com].set(W_s2).at[hid:, com:].set(W_p2)
    b2 = jnp.concatenate([b_s2, b_p2]).reshape(1, 2 * com)

    xp = jnp.pad(x, ((0, k_pad - n), (0, 0)))

    params = pltpu.CompilerParams(
        dimension_semantics=("parallel", "arbitrary"))

    p = pl.pallas_call(
        functools.partial(_layer1_kernel, bk=bk, n_total=n),
        grid=(grid_m, grid_k),
        in_specs=[
            pl.BlockSpec((bm, bk), lambda m, k: (m, k)),
            pl.BlockSpec((bk, ft), lambda m, k: (k, 0)),
            pl.BlockSpec((ft, 2 * hid), lambda m, k: (0, 0)),
            pl.BlockSpec((1, 2 * hid), lambda m, k: (0, 0)),
            pl.BlockSpec((2 * hid, 2 * com), lambda m, k: (0, 0)),
        ],
        out_specs=pl.BlockSpec((bm, 2 * com), lambda m, k: (m, 0)),
        out_shape=jax.ShapeDtypeStruct((n, 2 * com), jnp.float32),
        scratch_shapes=[pltpu.VMEM((bm, ft), jnp.float32)],
        compiler_params=params,
    )(adj, xp, w1, b1, w2)

    pp = jnp.pad(p, ((0, k_pad - n), (0, 0)))

    out = pl.pallas_call(
        functools.partial(_layer2_kernel, bk=bk, n_total=n),
        grid=(grid_m, grid_k),
        in_specs=[
            pl.BlockSpec((bm, bk), lambda m, k: (m, k)),
            pl.BlockSpec((bk, 2 * com), lambda m, k: (k, 0)),
            pl.BlockSpec((1, 2 * com), lambda m, k: (0, 0)),
        ],
        out_specs=pl.BlockSpec((bm, 2 * com), lambda m, k: (m, 0)),
        out_shape=jax.ShapeDtypeStruct((n, 2 * com), jnp.float32),
        scratch_shapes=[pltpu.VMEM((bm, 2 * com), jnp.float32)],
        compiler_params=params,
    )(adj, pp, b2)

    return out[:, :com], out[:, com:]


# speedup vs baseline: 1.2640x; 1.0875x over previous
import functools

import jax
import jax.numpy as jnp
from jax.experimental import pallas as pl
from jax.experimental.pallas import tpu as pltpu


def _layer1_kernel(adj_ref, x_ref, w1_ref, b1_ref, w2_ref, o_ref, acc_ref,
                   *, bk, n_total):
    k = pl.program_id(1)
    nk = pl.num_programs(1)

    @pl.when(k == 0)
    def _():
        acc_ref[...] = jnp.zeros_like(acc_ref)

    def acc_step(a):
        acc_ref[...] += jnp.dot(a.astype(jnp.bfloat16), x_ref[...],
                                preferred_element_type=jnp.float32)

    @pl.when(k < nk - 1)
    def _():
        acc_step(adj_ref[...])

    @pl.when(k == nk - 1)
    def _():
        a = adj_ref[...]
        cols = jax.lax.broadcasted_iota(jnp.int32, a.shape, 1)
        acc_step(jnp.where(cols < (n_total - k * bk), a, 0.0))
        z = acc_ref[...]
        h = jnp.dot(z.astype(jnp.bfloat16), w1_ref[...],
                    preferred_element_type=jnp.float32) + b1_ref[...]
        h = jnp.maximum(h, 0.0)
        o_ref[...] = jnp.dot(h.astype(jnp.bfloat16), w2_ref[...],
                             preferred_element_type=jnp.float32
                             ).astype(jnp.bfloat16)


def _layer2_kernel(adj_ref, p_ref, b_ref, o_ref, acc_ref, *, bk, n_total):
    k = pl.program_id(1)
    nk = pl.num_programs(1)

    @pl.when(k == 0)
    def _():
        acc_ref[...] = jnp.zeros_like(acc_ref)

    def acc_step(a):
        acc_ref[...] += jnp.dot(a.astype(jnp.bfloat16), p_ref[...],
                                preferred_element_type=jnp.float32)

    @pl.when(k < nk - 1)
    def _():
        acc_step(adj_ref[...])

    @pl.when(k == nk - 1)
    def _():
        a = adj_ref[...]
        cols = jax.lax.broadcasted_iota(jnp.int32, a.shape, 1)
        acc_step(jnp.where(cols < (n_total - k * bk), a, 0.0))
        o_ref[...] = acc_ref[...] + b_ref[...]


def kernel(x, adj, W_s1, b_s1, W_s2, b_s2, W_p1, b_p1, W_p2, b_p2,
           enc_mask_token):
    del enc_mask_token
    n, ft = x.shape
    hid = W_s1.shape[1]
    com = W_s2.shape[1]

    bm = 400
    bk = 2048
    grid_m = pl.cdiv(n, bm)
    grid_k = pl.cdiv(n, bk)
    k_pad = grid_k * bk

    w1 = jnp.concatenate([W_s1, W_p1], axis=1).astype(jnp.bfloat16)
    b1 = jnp.concatenate([b_s1, b_p1]).reshape(1, 2 * hid)
    w2 = jnp.zeros((2 * hid, 2 * com), jnp.float32)
    w2 = w2.at[:hid, :com].set(W_s2).at[hid:, com:].set(W_p2)
    w2 = w2.astype(jnp.bfloat16)
    b2 = jnp.concatenate([b_s2, b_p2]).reshape(1, 2 * com)

    xp = jnp.pad(x, ((0, k_pad - n), (0, 0))).astype(jnp.bfloat16)

    params = pltpu.CompilerParams(
        dimension_semantics=("parallel", "arbitrary"))

    p = pl.pallas_call(
        functools.partial(_layer1_kernel, bk=bk, n_total=n),
        grid=(grid_m, grid_k),
        in_specs=[
            pl.BlockSpec((bm, bk), lambda m, k: (m, k)),
            pl.BlockSpec((bk, ft), lambda m, k: (k, 0)),
            pl.BlockSpec((ft, 2 * hid), lambda m, k: (0, 0)),
            pl.BlockSpec((1, 2 * hid), lambda m, k: (0, 0)),
            pl.BlockSpec((2 * hid, 2 * com), lambda m, k: (0, 0)),
        ],
        out_specs=pl.BlockSpec((bm, 2 * com), lambda m, k: (m, 0)),
        out_shape=jax.ShapeDtypeStruct((n, 2 * com), jnp.bfloat16),
        scratch_shapes=[pltpu.VMEM((bm, ft), jnp.float32)],
        compiler_params=params,
    )(adj, xp, w1, b1, w2)

    pp = jnp.pad(p, ((0, k_pad - n), (0, 0)))

    out = pl.pallas_call(
        functools.partial(_layer2_kernel, bk=bk, n_total=n),
        grid=(grid_m, grid_k),
        in_specs=[
            pl.BlockSpec((bm, bk), lambda m, k: (m, k)),
            pl.BlockSpec((bk, 2 * com), lambda m, k: (k, 0)),
            pl.BlockSpec((1, 2 * com), lambda m, k: (0, 0)),
        ],
        out_specs=pl.BlockSpec((bm, 2 * com), lambda m, k: (m, 0)),
        out_shape=jax.ShapeDtypeStruct((n, 2 * com), jnp.float32),
        scratch_shapes=[pltpu.VMEM((bm, 2 * com), jnp.float32)],
        compiler_params=params,
    )(adj, pp, b2)

    return out[:, :com], out[:, com:]


# speedup vs baseline: 1.8560x; 1.4684x over previous
import functools

import jax
import jax.numpy as jnp
from jax.experimental import pallas as pl
from jax.experimental.pallas import tpu as pltpu


def _body(a, b_ref, k0, kt_valid):
    z = jnp.dot(a[:, :k0], b_ref[:k0, :], preferred_element_type=jnp.float32)
    tail = a[:, k0:]
    cols = jax.lax.broadcasted_iota(jnp.int32, tail.shape, 1)
    tail = jnp.where(cols < kt_valid, tail, 0.0)
    z += jnp.dot(tail, b_ref[k0:, :], preferred_element_type=jnp.float32)
    return z


def _layer1_kernel(adj_ref, x_ref, w1_ref, b1_ref, w2_ref, o_ref,
                   *, k0, kt_valid):
    z = _body(adj_ref[...], x_ref, k0, kt_valid)
    h = jnp.dot(z.astype(jnp.bfloat16), w1_ref[...],
                preferred_element_type=jnp.float32) + b1_ref[...]
    h = jnp.maximum(h, 0.0)
    o_ref[...] = jnp.dot(h.astype(jnp.bfloat16), w2_ref[...],
                         preferred_element_type=jnp.float32
                         ).astype(jnp.bfloat16)


def _layer2_kernel(adj_ref, p_ref, b_ref, o_ref, *, k0, kt_valid):
    z = _body(adj_ref[...], p_ref, k0, kt_valid)
    o_ref[...] = z + b_ref[...]


def kernel(x, adj, W_s1, b_s1, W_s2, b_s2, W_p1, b_p1, W_p2, b_p2,
           enc_mask_token):
    del enc_mask_token
    n, ft = x.shape
    hid = W_s1.shape[1]
    com = W_s2.shape[1]

    bm = 400
    grid_m = pl.cdiv(n, bm)
    k_pad = ((n + 1023) // 1024) * 1024
    k0 = (n // 128) * 128
    kt_valid = n - k0

    w1 = jnp.concatenate([W_s1, W_p1], axis=1).astype(jnp.bfloat16)
    b1 = jnp.concatenate([b_s1, b_p1]).reshape(1, 2 * hid)
    w2 = jnp.zeros((2 * hid, 2 * com), jnp.float32)
    w2 = w2.at[:hid, :com].set(W_s2).at[hid:, com:].set(W_p2)
    w2 = w2.astype(jnp.bfloat16)
    b2 = jnp.concatenate([b_s2, b_p2]).reshape(1, 2 * com)

    xp = jnp.pad(x, ((0, k_pad - n), (0, 0))).astype(jnp.bfloat16)

    params = pltpu.CompilerParams(dimension_semantics=("parallel",))

    p = pl.pallas_call(
        functools.partial(_layer1_kernel, k0=k0, kt_valid=kt_valid),
        grid=(grid_m,),
        in_specs=[
            pl.BlockSpec((bm, k_pad), lambda m: (m, 0)),
            pl.BlockSpec((k_pad, ft), lambda m: (0, 0)),
            pl.BlockSpec((ft, 2 * hid), lambda m: (0, 0)),
            pl.BlockSpec((1, 2 * hid), lambda m: (0, 0)),
            pl.BlockSpec((2 * hid, 2 * com), lambda m: (0, 0)),
        ],
        out_specs=pl.BlockSpec((bm, 2 * com), lambda m: (m, 0)),
        out_shape=jax.ShapeDtypeStruct((n, 2 * com), jnp.bfloat16),
        compiler_params=params,
    )(adj, xp, w1, b1, w2)

    pp = jnp.pad(p, ((0, k_pad - n), (0, 0)))

    out = pl.pallas_call(
        functools.partial(_layer2_kernel, k0=k0, kt_valid=kt_valid),
        grid=(grid_m,),
        in_specs=[
            pl.BlockSpec((bm, k_pad), lambda m: (m, 0)),
            pl.BlockSpec((k_pad, 2 * com), lambda m: (0, 0)),
            pl.BlockSpec((1, 2 * com), lambda m: (0, 0)),
        ],
        out_specs=pl.BlockSpec((bm, 2 * com), lambda m: (m, 0)),
        out_shape=jax.ShapeDtypeStruct((n, 2 * com), jnp.float32),
        compiler_params=params,
    )(adj, pp, b2)

    return out[:, :com], out[:, com:]


# speedup vs baseline: 1.9375x; 1.0439x over previous
import functools

import jax
import jax.numpy as jnp
from jax.experimental import pallas as pl
from jax.experimental.pallas import tpu as pltpu


def _masked_tail(t, axis, valid):
    idx = jax.lax.broadcasted_iota(jnp.int32, t.shape, axis)
    return jnp.where(idx < valid, t, jnp.zeros_like(t))


def _spmm(adj_ref, b_ref, k0, kt_valid):
    a = adj_ref[...]
    z = jnp.dot(a[:, :k0], b_ref[:k0, :], preferred_element_type=jnp.float32)
    a_tail = _masked_tail(a[:, k0:], 1, kt_valid)
    b_tail = _masked_tail(b_ref[k0:, :], 0, kt_valid)
    return z + jnp.dot(a_tail, b_tail, preferred_element_type=jnp.float32)


def _layer1_kernel(adj_ref, x_ref, w1_ref, b1_ref, w2_ref, o_ref,
                   *, k0, kt_valid):
    z = _spmm(adj_ref, x_ref, k0, kt_valid)
    h = jnp.dot(z.astype(jnp.bfloat16), w1_ref[...],
                preferred_element_type=jnp.float32) + b1_ref[...]
    h = jnp.maximum(h, 0.0)
    o_ref[...] = jnp.dot(h.astype(jnp.bfloat16), w2_ref[...],
                         preferred_element_type=jnp.float32
                         ).astype(jnp.bfloat16)


def _layer2_kernel(adj_ref, p_ref, b_ref, oc_ref, op_ref,
                   *, k0, kt_valid, com):
    z = _spmm(adj_ref, p_ref, k0, kt_valid) + b_ref[...]
    oc_ref[...] = z[:, :com]
    op_ref[...] = z[:, com:]


def kernel(x, adj, W_s1, b_s1, W_s2, b_s2, W_p1, b_p1, W_p2, b_p2,
           enc_mask_token):
    del enc_mask_token
    n, ft = x.shape
    hid = W_s1.shape[1]
    com = W_s2.shape[1]

    bm = 400
    grid_m = pl.cdiv(n, bm)
    k_pad = ((n + 1023) // 1024) * 1024
    k0 = (n // 128) * 128
    kt_valid = n - k0

    w1 = jnp.concatenate([W_s1, W_p1], axis=1).astype(jnp.bfloat16)
    b1 = jnp.concatenate([b_s1, b_p1]).reshape(1, 2 * hid)
    w2 = jnp.zeros((2 * hid, 2 * com), jnp.float32)
    w2 = w2.at[:hid, :com].set(W_s2).at[hid:, com:].set(W_p2)
    w2 = w2.astype(jnp.bfloat16)
    b2 = jnp.concatenate([b_s2, b_p2]).reshape(1, 2 * com)
    xb = x.astype(jnp.bfloat16)

    params = pltpu.CompilerParams(dimension_semantics=("arbitrary",))

    p = pl.pallas_call(
        functools.partial(_layer1_kernel, k0=k0, kt_valid=kt_valid),
        grid=(grid_m,),
        in_specs=[
            pl.BlockSpec((bm, k_pad), lambda m: (m, 0)),
            pl.BlockSpec((k_pad, ft), lambda m: (0, 0)),
            pl.BlockSpec((ft, 2 * hid), lambda m: (0, 0)),
            pl.BlockSpec((1, 2 * hid), lambda m: (0, 0)),
            pl.BlockSpec((2 * hid, 2 * com), lambda m: (0, 0)),
        ],
        out_specs=pl.BlockSpec((bm, 2 * com), lambda m: (m, 0)),
        out_shape=jax.ShapeDtypeStruct((n, 2 * com), jnp.bfloat16),
        compiler_params=params,
    )(adj, xb, w1, b1, w2)

    out_c, out_p = pl.pallas_call(
        functools.partial(_layer2_kernel, k0=k0, kt_valid=kt_valid, com=com),
        grid=(grid_m,),
        in_specs=[
            pl.BlockSpec((bm, k_pad), lambda m: (m, 0)),
            pl.BlockSpec((k_pad, 2 * com), lambda m: (0, 0)),
            pl.BlockSpec((1, 2 * com), lambda m: (0, 0)),
        ],
        out_specs=[
            pl.BlockSpec((bm, com), lambda m: (m, 0)),
            pl.BlockSpec((bm, com), lambda m: (m, 0)),
        ],
        out_shape=[
            jax.ShapeDtypeStruct((n, com), jnp.float32),
            jax.ShapeDtypeStruct((n, com), jnp.float32),
        ],
        compiler_params=params,
    )(adj, p, b2)

    return out_c, out_p
